# baseline (speedup 1.0000x reference)
import functools

import jax
import jax.numpy as jnp
from jax import lax
from jax.experimental import pallas as pl
from jax.experimental.pallas import tpu as pltpu
from jax.experimental.pallas import tpu_sc as plsc

B = 4
H = 224
W = 224
NPIX = B * H * W
NTILES = 32
PIX_PER_TILE = NPIX // NTILES
ROWS_PER_TILE = PIX_PER_TILE // W
CHUNK = W // 2
NCHUNK = PIX_PER_TILE // CHUNK
VEC = 16


def _warp_body(table_hbm, dx_hbm, dy_hbm, out_hbm, dx_v, dy_v, idx_v, rows_v, sem):
    wid = lax.axis_index("s") * 2 + lax.axis_index("c")
    base = wid * PIX_PER_TILE
    pltpu.sync_copy(dx_hbm.at[pl.ds(base, PIX_PER_TILE)], dx_v)
    pltpu.sync_copy(dy_hbm.at[pl.ds(base, PIX_PER_TILE)], dy_v)

    lane = lax.iota(jnp.int32, VEC)
    bb = wid >> 3
    i0 = (wid & 7) * ROWS_PER_TILE
    bf = bb.astype(jnp.float32)
    g0 = jnp.minimum(lane, 3) * (H * W)

    def row_body(r, carry):
        roff = r * W
        fi = (i0 + r).astype(jnp.float32)
        for v in range(W // VEC):
            off = roff + v * VEC
            fy = bf + dy_v[pl.ds(off, VEC)]
            fx = fi + dx_v[pl.ds(off, VEC)]
            yy = jnp.minimum(jnp.mod(fy, 224.0).astype(jnp.int32), H - 1)
            xx = jnp.minimum(jnp.mod(fx, 224.0).astype(jnp.int32), W - 1)
            goff = g0 if v == 0 else 3 * (H * W)
            c, s = (2 * r, v * VEC) if v < 7 else (2 * r + 1, v * VEC - CHUNK)
            idx_v[c, pl.ds(s, VEC)] = goff + yy * W + xx
        d1 = pltpu.async_copy(
            table_hbm.at[idx_v.at[2 * r]],
            rows_v.at[pl.ds(roff, CHUNK)],
            sem,
        )
        d2 = pltpu.async_copy(
            table_hbm.at[idx_v.at[2 * r + 1]],
            rows_v.at[pl.ds(roff + CHUNK, CHUNK)],
            sem,
        )
        d1.wait()
        d2.wait()
        return carry

    lax.fori_loop(0, ROWS_PER_TILE, row_body, 0)
    pltpu.sync_copy(rows_v, out_hbm.at[pl.ds(base, PIX_PER_TILE)])


_warp = functools.partial(
    pl.kernel,
    out_type=jax.ShapeDtypeStruct((NPIX, 3), jnp.float32),
    mesh=plsc.VectorSubcoreMesh(core_axis_name="c", subcore_axis_name="s"),
    scratch_types=[
        pltpu.VMEM((PIX_PER_TILE,), jnp.float32),
        pltpu.VMEM((PIX_PER_TILE,), jnp.float32),
        pltpu.VMEM((NCHUNK, CHUNK), jnp.int32),
        pltpu.VMEM((PIX_PER_TILE, 3), jnp.float32),
        pltpu.SemaphoreType.DMA,
    ],
    compiler_params=pltpu.CompilerParams(use_tc_tiling_on_sc=False),
)(_warp_body)


def kernel(x):
    table = x[..., :3].reshape(NPIX, 3)
    dxf = x[..., 3].reshape(NPIX)
    dyf = x[..., 4].reshape(NPIX)
    out = _warp(table, dxf, dyf)
    return out.reshape(B, H, W, 3)
